# baseline (speedup 1.0000x reference)
import functools

import jax
import jax.numpy as jnp
from jax import lax
from jax.experimental import pallas as pl
from jax.experimental.pallas import tpu as pltpu
from jax.experimental.pallas import tpu_sc as plsc

ROUTER_N_ITERS = 20
ROUTER_EPS = 0.03
ROUTER_EPS_INIT = 4.0
ROUTER_EPS_DECAY = 0.7
ROUTER_FETCH_K_RATIO = 9.0 / 8.0

XB, XN, XD = 4, 8192, 1024
NROWS = XB * XN
MV_BLK = 4096
TOPK = 1024

NW = 32
BSEL = XB * TOPK
BPW = BSEL // NW
CH = 32
NCH = BPW // CH


def _eps_schedule():
    eps_list = []
    cur = max(ROUTER_EPS_INIT, ROUTER_EPS)
    for _ in range(ROUTER_N_ITERS):
        eps_list.append(cur)
        cur = max(cur * ROUTER_EPS_DECAY, ROUTER_EPS)
    return eps_list, cur


def _mv_body(x_ref, rt_ref, o_ref):
    o_ref[0] = lax.dot_general(rt_ref[...], x_ref[...],
                               (((1,), (1,)), ((), ())))


NSEG = XN // TOPK


def _cmp_exchange(key, idx, p_key, p_idx, i_is_lo, dir_up):
    cur_first = (key > p_key) | ((key == p_key) & (idx < p_idx))
    take_cur = (cur_first == i_is_lo) == dir_up
    return jnp.where(take_cur, key, p_key), jnp.where(take_cur, idx, p_idx)


def _lane_stage(key, idx, j, dir_up, lane_tiebreak=False):
    pos = lax.broadcasted_iota(jnp.int32, key.shape, 1)
    bitj = (pos & j) != 0
    p_key = jnp.where(bitj, jnp.roll(key, j, axis=-1),
                      jnp.roll(key, -j, axis=-1))
    p_idx = jnp.where(bitj, jnp.roll(idx, j, axis=-1),
                      jnp.roll(idx, -j, axis=-1))
    if lane_tiebreak:
        cur_first = (key > p_key) | ((key == p_key) & (~bitj))
        take_cur = (cur_first == (~bitj)) == dir_up
        return (jnp.where(take_cur, key, p_key),
                jnp.where(take_cur, idx, p_idx))
    return _cmp_exchange(key, idx, p_key, p_idx, ~bitj, dir_up)


def _router_body(s_ref, nt_ref, idx_ref, sc_ref, gidx_ref):
    s = s_ref[...]
    ek = jnp.minimum(nt_ref[0, 0].astype(jnp.float32)
                     * jnp.float32(ROUTER_FETCH_K_RATIO), jnp.float32(XN))
    logk = jnp.log(jnp.maximum(ek, jnp.float32(1e-20)))
    b = -s
    eps_list, eps_fin = _eps_schedule()
    a = jnp.zeros((XB, 1, 1), jnp.float32)
    for eps in eps_list:
        sb = (s + b) / eps
        m = jnp.max(sb, axis=(1, 2), keepdims=True)
        lse = jnp.log(jnp.sum(jnp.exp(sb - m), axis=(1, 2), keepdims=True)) + m
        a = eps * (logk - lse)
        b = -jnp.maximum(s + a, 0.0)
    scores = jnp.exp((s + a + b) / eps_fin)

    nrows = XB * NSEG
    key = scores.reshape(nrows, TOPK)
    rowi = lax.broadcasted_iota(jnp.int32, (nrows, TOPK), 0)
    lane = lax.broadcasted_iota(jnp.int32, (nrows, TOPK), 1)
    idx = (rowi & (NSEG - 1)) * TOPK + lane
    row_odd = (rowi & 1) != 0
    pos = lane
    k = 2
    while k <= TOPK:
        j = k // 2
        while j >= 1:
            dir_up = ((pos & k) == 0) != row_odd
            key, idx = _lane_stage(key, idx, j, dir_up)
            j //= 2
        k *= 2

    r = nrows
    while r > XB:
        k4 = key.reshape(r // 2, 2, TOPK)
        i4 = idx.reshape(r // 2, 2, TOPK)
        ak, bk = k4[:, 0, :], k4[:, 1, :]
        ai, bi = i4[:, 0, :], i4[:, 1, :]
        afirst = (ak > bk) | ((ak == bk) & (ai < bi))
        key = jnp.where(afirst, ak, bk)
        idx = jnp.where(afirst, ai, bi)
        r //= 2
        if r > XB:
            rowi = lax.broadcasted_iota(jnp.int32, (r, TOPK), 0)
            dir_up = (rowi & 1) == 0
        else:
            dir_up = True
        pos = lax.broadcasted_iota(jnp.int32, (r, TOPK), 1)
        j = TOPK // 2
        while j >= 1:
            key, idx = _lane_stage(key, idx, j, dir_up)
            j //= 2

    sel_v = key
    sel_i = idx
    idx_ref[...] = sel_i
    sc_ref[...] = sel_v + (1.0 - sel_v)
    row = lax.broadcasted_iota(jnp.int32, (XB, TOPK), 0)
    gidx_ref[...] = row * XN + sel_i


def _router_call(s, nt_arr):
    return pl.pallas_call(
        _router_body,
        in_specs=[pl.BlockSpec(memory_space=pltpu.VMEM),
                  pl.BlockSpec(memory_space=pltpu.SMEM)],
        out_shape=(jax.ShapeDtypeStruct((XB, TOPK), jnp.int32),
                   jax.ShapeDtypeStruct((XB, TOPK), jnp.float32),
                   jax.ShapeDtypeStruct((XB, TOPK), jnp.int32)),
    )(s.reshape(XB, NSEG, TOPK), nt_arr)


def _sc_gather(table, gidx):
    mesh = plsc.VectorSubcoreMesh(core_axis_name="c", subcore_axis_name="s")

    @functools.partial(
        pl.kernel, mesh=mesh,
        out_type=jax.ShapeDtypeStruct((BSEL, XD), jnp.float32),
        scratch_types=[pltpu.VMEM((BPW,), jnp.int32),
                       pltpu.VMEM((CH, XD), jnp.float32),
                       pltpu.VMEM((CH, XD), jnp.float32),
                       pltpu.SemaphoreType.DMA,
                       pltpu.SemaphoreType.DMA],
    )
    def gk(idx_hbm, table_hbm, out_hbm, idx_v, rows_v0, rows_v1, sem0, sem1):
        cid = lax.axis_index("c")
        sid = lax.axis_index("s")
        wid = sid * 2 + cid
        base = wid * BPW
        bufs = (rows_v0, rows_v1)
        sems = (sem0, sem1)
        pltpu.sync_copy(idx_hbm.at[wid // (TOPK // BPW), pl.ds((wid % (TOPK // BPW)) * BPW, BPW)], idx_v)
        cur = pltpu.async_copy(table_hbm.at[idx_v.at[pl.ds(0, CH)]], bufs[0], sems[0])
        for c in range(NCH):
            nxt = None
            if c + 1 < NCH:
                nxt = pltpu.async_copy(table_hbm.at[idx_v.at[pl.ds((c + 1) * CH, CH)]],
                                       bufs[(c + 1) % 2], sems[(c + 1) % 2])
            cur.wait()
            pltpu.sync_copy(bufs[c % 2], out_hbm.at[pl.ds(base + c * CH, CH)])
            cur = nxt

    return gk(gidx, table)


def kernel(x, routing_token, num_tokens):
    b, n, d = x.shape
    x2 = x.reshape(NROWS, XD)
    nblk = NROWS // MV_BLK
    s3 = pl.pallas_call(
        _mv_body,
        grid=(nblk,),
        in_specs=[pl.BlockSpec((MV_BLK, XD), lambda i: (i, 0)),
                  pl.BlockSpec((1, XD), lambda i: (0, 0))],
        out_specs=pl.BlockSpec((1, 1, MV_BLK), lambda i: (i, 0, 0)),
        out_shape=jax.ShapeDtypeStruct((nblk, 1, MV_BLK), jnp.float32),
    )(x2, routing_token)
    s = s3.reshape(b, n)

    nt_arr = jnp.asarray(num_tokens, jnp.int32).reshape(1, 1)
    sel_i, sel_sc, gidx = _router_call(s, nt_arr)
    routed = _sc_gather(x2, gidx).reshape(b, TOPK, XD)
    return sel_i, sel_sc, routed


# speedup vs baseline: 1.0162x; 1.0162x over previous
import functools

import jax
import jax.numpy as jnp
from jax import lax
from jax.experimental import pallas as pl
from jax.experimental.pallas import tpu as pltpu
from jax.experimental.pallas import tpu_sc as plsc

ROUTER_N_ITERS = 20
ROUTER_EPS = 0.03
ROUTER_EPS_INIT = 4.0
ROUTER_EPS_DECAY = 0.7
ROUTER_FETCH_K_RATIO = 9.0 / 8.0

XB, XN, XD = 4, 8192, 1024
NROWS = XB * XN
MV_BLK = 2048
TOPK = 1024

NW = 32
BSEL = XB * TOPK
BPW = BSEL // NW
CH = 32
NCH = BPW // CH


def _eps_schedule():
    eps_list = []
    cur = max(ROUTER_EPS_INIT, ROUTER_EPS)
    for _ in range(ROUTER_N_ITERS):
        eps_list.append(cur)
        cur = max(cur * ROUTER_EPS_DECAY, ROUTER_EPS)
    return eps_list, cur


def _mv_body(x_ref, rt_ref, o_ref):
    o_ref[0] = lax.dot_general(rt_ref[...], x_ref[...],
                               (((1,), (1,)), ((), ())))


NSEG = XN // TOPK


def _cmp_exchange(key, idx, p_key, p_idx, i_is_lo, dir_up):
    cur_first = (key > p_key) | ((key == p_key) & (idx < p_idx))
    take_cur = (cur_first == i_is_lo) == dir_up
    return jnp.where(take_cur, key, p_key), jnp.where(take_cur, idx, p_idx)


def _lane_stage(key, idx, j, dir_up, lane_tiebreak=False):
    pos = lax.broadcasted_iota(jnp.int32, key.shape, 1)
    bitj = (pos & j) != 0
    p_key = jnp.where(bitj, jnp.roll(key, j, axis=-1),
                      jnp.roll(key, -j, axis=-1))
    p_idx = jnp.where(bitj, jnp.roll(idx, j, axis=-1),
                      jnp.roll(idx, -j, axis=-1))
    if lane_tiebreak:
        cur_first = (key > p_key) | ((key == p_key) & (~bitj))
        take_cur = (cur_first == (~bitj)) == dir_up
        return (jnp.where(take_cur, key, p_key),
                jnp.where(take_cur, idx, p_idx))
    return _cmp_exchange(key, idx, p_key, p_idx, ~bitj, dir_up)


def _router_body(s_ref, nt_ref, idx_ref, sc_ref, gidx_ref):
    s = s_ref[...]
    ek = jnp.minimum(nt_ref[0, 0].astype(jnp.float32)
                     * jnp.float32(ROUTER_FETCH_K_RATIO), jnp.float32(XN))
    logk = jnp.log(jnp.maximum(ek, jnp.float32(1e-20)))
    b = -s
    eps_list, eps_fin = _eps_schedule()
    a = jnp.zeros((XB, 1, 1), jnp.float32)
    for eps in eps_list:
        sb = (s + b) / eps
        m = jnp.max(sb, axis=(1, 2), keepdims=True)
        lse = jnp.log(jnp.sum(jnp.exp(sb - m), axis=(1, 2), keepdims=True)) + m
        a = eps * (logk - lse)
        b = -jnp.maximum(s + a, 0.0)
    scores = jnp.exp((s + a + b) / eps_fin)

    nrows = XB * NSEG
    key = scores.reshape(nrows, TOPK)
    rowi = lax.broadcasted_iota(jnp.int32, (nrows, TOPK), 0)
    lane = lax.broadcasted_iota(jnp.int32, (nrows, TOPK), 1)
    idx = (rowi & (NSEG - 1)) * TOPK + lane
    row_odd = (rowi & 1) != 0
    pos = lane
    k = 2
    while k <= TOPK:
        j = k // 2
        while j >= 1:
            dir_up = ((pos & k) == 0) != row_odd
            key, idx = _lane_stage(key, idx, j, dir_up)
            j //= 2
        k *= 2

    r = nrows
    while r > XB:
        k4 = key.reshape(r // 2, 2, TOPK)
        i4 = idx.reshape(r // 2, 2, TOPK)
        ak, bk = k4[:, 0, :], k4[:, 1, :]
        ai, bi = i4[:, 0, :], i4[:, 1, :]
        afirst = (ak > bk) | ((ak == bk) & (ai < bi))
        key = jnp.where(afirst, ak, bk)
        idx = jnp.where(afirst, ai, bi)
        r //= 2
        if r > XB:
            rowi = lax.broadcasted_iota(jnp.int32, (r, TOPK), 0)
            dir_up = (rowi & 1) == 0
        else:
            dir_up = True
        pos = lax.broadcasted_iota(jnp.int32, (r, TOPK), 1)
        j = TOPK // 2
        while j >= 1:
            key, idx = _lane_stage(key, idx, j, dir_up)
            j //= 2

    sel_v = key
    sel_i = idx
    idx_ref[...] = sel_i
    sc_ref[...] = sel_v + (1.0 - sel_v)
    row = lax.broadcasted_iota(jnp.int32, (XB, TOPK), 0)
    gidx_ref[...] = row * XN + sel_i


def _router_call(s, nt_arr):
    return pl.pallas_call(
        _router_body,
        in_specs=[pl.BlockSpec(memory_space=pltpu.VMEM),
                  pl.BlockSpec(memory_space=pltpu.SMEM)],
        out_shape=(jax.ShapeDtypeStruct((XB, TOPK), jnp.int32),
                   jax.ShapeDtypeStruct((XB, TOPK), jnp.float32),
                   jax.ShapeDtypeStruct((XB, TOPK), jnp.int32)),
    )(s.reshape(XB, NSEG, TOPK), nt_arr)


def _sc_gather(table, gidx):
    mesh = plsc.VectorSubcoreMesh(core_axis_name="c", subcore_axis_name="s")

    @functools.partial(
        pl.kernel, mesh=mesh,
        out_type=jax.ShapeDtypeStruct((BSEL, XD), jnp.float32),
        scratch_types=[pltpu.VMEM((BPW,), jnp.int32),
                       pltpu.VMEM((CH, XD), jnp.float32),
                       pltpu.VMEM((CH, XD), jnp.float32),
                       pltpu.SemaphoreType.DMA,
                       pltpu.SemaphoreType.DMA],
    )
    def gk(idx_hbm, table_hbm, out_hbm, idx_v, rows_v0, rows_v1, sem0, sem1):
        cid = lax.axis_index("c")
        sid = lax.axis_index("s")
        wid = sid * 2 + cid
        base = wid * BPW
        bufs = (rows_v0, rows_v1)
        sems = (sem0, sem1)
        pltpu.sync_copy(idx_hbm.at[wid // (TOPK // BPW), pl.ds((wid % (TOPK // BPW)) * BPW, BPW)], idx_v)
        cur = pltpu.async_copy(table_hbm.at[idx_v.at[pl.ds(0, CH)]], bufs[0], sems[0])
        for c in range(NCH):
            nxt = None
            if c + 1 < NCH:
                nxt = pltpu.async_copy(table_hbm.at[idx_v.at[pl.ds((c + 1) * CH, CH)]],
                                       bufs[(c + 1) % 2], sems[(c + 1) % 2])
            cur.wait()
            pltpu.sync_copy(bufs[c % 2], out_hbm.at[pl.ds(base + c * CH, CH)])
            cur = nxt

    return gk(gidx, table)


def kernel(x, routing_token, num_tokens):
    b, n, d = x.shape
    x2 = x.reshape(NROWS, XD)
    nblk = NROWS // MV_BLK
    s3 = pl.pallas_call(
        _mv_body,
        grid=(nblk,),
        in_specs=[pl.BlockSpec((MV_BLK, XD), lambda i: (i, 0)),
                  pl.BlockSpec((1, XD), lambda i: (0, 0))],
        out_specs=pl.BlockSpec((1, 1, MV_BLK), lambda i: (i, 0, 0)),
        out_shape=jax.ShapeDtypeStruct((nblk, 1, MV_BLK), jnp.float32),
    )(x2, routing_token)
    s = s3.reshape(b, n)

    nt_arr = jnp.asarray(num_tokens, jnp.int32).reshape(1, 1)
    sel_i, sel_sc, gidx = _router_call(s, nt_arr)
    routed = _sc_gather(x2, gidx).reshape(b, TOPK, XD)
    return sel_i, sel_sc, routed


# speedup vs baseline: 1.0210x; 1.0047x over previous
import functools

import jax
import jax.numpy as jnp
from jax import lax
from jax.experimental import pallas as pl
from jax.experimental.pallas import tpu as pltpu
from jax.experimental.pallas import tpu_sc as plsc

ROUTER_N_ITERS = 20
ROUTER_EPS = 0.03
ROUTER_EPS_INIT = 4.0
ROUTER_EPS_DECAY = 0.7
ROUTER_FETCH_K_RATIO = 9.0 / 8.0

XB, XN, XD = 4, 8192, 1024
NROWS = XB * XN
MV_BLK = 2048
TOPK = 1024

NW = 32
BSEL = XB * TOPK
BPW = BSEL // NW
CH = 32
NCH = BPW // CH


def _eps_schedule():
    eps_list = []
    cur = max(ROUTER_EPS_INIT, ROUTER_EPS)
    for _ in range(ROUTER_N_ITERS):
        eps_list.append(cur)
        cur = max(cur * ROUTER_EPS_DECAY, ROUTER_EPS)
    return eps_list, cur


def _mv_body(x_ref, rt_ref, o_ref):
    o_ref[0] = lax.dot_general(rt_ref[...], x_ref[...],
                               (((1,), (1,)), ((), ())))


NSEG = XN // TOPK


def _cmp_exchange(key, idx, p_key, p_idx, i_is_lo, dir_up):
    cur_first = (key > p_key) | ((key == p_key) & (idx < p_idx))
    take_cur = (cur_first == i_is_lo) == dir_up
    return jnp.where(take_cur, key, p_key), jnp.where(take_cur, idx, p_idx)


def _lane_stage(key, idx, j, dir_up, lane_tiebreak=False):
    pos = lax.broadcasted_iota(jnp.int32, key.shape, 1)
    bitj = (pos & j) != 0
    p_key = jnp.where(bitj, jnp.roll(key, j, axis=-1),
                      jnp.roll(key, -j, axis=-1))
    p_idx = jnp.where(bitj, jnp.roll(idx, j, axis=-1),
                      jnp.roll(idx, -j, axis=-1))
    if lane_tiebreak:
        cur_first = (key > p_key) | ((key == p_key) & (~bitj))
        take_cur = (cur_first == (~bitj)) == dir_up
        return (jnp.where(take_cur, key, p_key),
                jnp.where(take_cur, idx, p_idx))
    return _cmp_exchange(key, idx, p_key, p_idx, ~bitj, dir_up)


def _router_body(s_ref, nt_ref, idx_ref, sc_ref, gidx_ref):
    s = s_ref[...]
    ek = jnp.minimum(nt_ref[0, 0].astype(jnp.float32)
                     * jnp.float32(ROUTER_FETCH_K_RATIO), jnp.float32(XN))
    logk = jnp.log(jnp.maximum(ek, jnp.float32(1e-20)))
    eps_list, eps_fin = _eps_schedule()
    a = eps_list[0] * (logk - jnp.log(jnp.float32(XN)))
    b = -jnp.maximum(s + a, 0.0)
    for eps in eps_list[1:]:
        sb = (s + b) / eps
        m = jnp.max(sb, axis=(1, 2), keepdims=True)
        lse = jnp.log(jnp.sum(jnp.exp(sb - m), axis=(1, 2), keepdims=True)) + m
        a = eps * (logk - lse)
        b = -jnp.maximum(s + a, 0.0)
    scores = jnp.exp((s + a + b) / eps_fin)

    nrows = XB * NSEG
    key = scores.reshape(nrows, TOPK)
    rowi = lax.broadcasted_iota(jnp.int32, (nrows, TOPK), 0)
    lane = lax.broadcasted_iota(jnp.int32, (nrows, TOPK), 1)
    idx = (rowi & (NSEG - 1)) * TOPK + lane
    row_odd = (rowi & 1) != 0
    pos = lane
    k = 2
    while k <= TOPK:
        j = k // 2
        while j >= 1:
            dir_up = ((pos & k) == 0) != row_odd
            key, idx = _lane_stage(key, idx, j, dir_up)
            j //= 2
        k *= 2

    r = nrows
    while r > XB:
        k4 = key.reshape(r // 2, 2, TOPK)
        i4 = idx.reshape(r // 2, 2, TOPK)
        ak, bk = k4[:, 0, :], k4[:, 1, :]
        ai, bi = i4[:, 0, :], i4[:, 1, :]
        afirst = (ak > bk) | ((ak == bk) & (ai < bi))
        key = jnp.where(afirst, ak, bk)
        idx = jnp.where(afirst, ai, bi)
        r //= 2
        if r > XB:
            rowi = lax.broadcasted_iota(jnp.int32, (r, TOPK), 0)
            dir_up = (rowi & 1) == 0
        else:
            dir_up = True
        pos = lax.broadcasted_iota(jnp.int32, (r, TOPK), 1)
        j = TOPK // 2
        while j >= 1:
            key, idx = _lane_stage(key, idx, j, dir_up)
            j //= 2

    sel_v = key
    sel_i = idx
    idx_ref[...] = sel_i
    sc_ref[...] = sel_v + (1.0 - sel_v)
    row = lax.broadcasted_iota(jnp.int32, (XB, TOPK), 0)
    gidx_ref[...] = row * XN + sel_i


def _router_call(s, nt_arr):
    return pl.pallas_call(
        _router_body,
        in_specs=[pl.BlockSpec(memory_space=pltpu.VMEM),
                  pl.BlockSpec(memory_space=pltpu.SMEM)],
        out_shape=(jax.ShapeDtypeStruct((XB, TOPK), jnp.int32),
                   jax.ShapeDtypeStruct((XB, TOPK), jnp.float32),
                   jax.ShapeDtypeStruct((XB, TOPK), jnp.int32)),
    )(s.reshape(XB, NSEG, TOPK), nt_arr)


def _sc_gather(table, gidx):
    mesh = plsc.VectorSubcoreMesh(core_axis_name="c", subcore_axis_name="s")

    @functools.partial(
        pl.kernel, mesh=mesh,
        out_type=jax.ShapeDtypeStruct((BSEL, XD), jnp.float32),
        scratch_types=[pltpu.VMEM((BPW,), jnp.int32),
                       pltpu.VMEM((CH, XD), jnp.float32),
                       pltpu.VMEM((CH, XD), jnp.float32),
                       pltpu.SemaphoreType.DMA,
                       pltpu.SemaphoreType.DMA],
    )
    def gk(idx_hbm, table_hbm, out_hbm, idx_v, rows_v0, rows_v1, sem0, sem1):
        cid = lax.axis_index("c")
        sid = lax.axis_index("s")
        wid = sid * 2 + cid
        base = wid * BPW
        bufs = (rows_v0, rows_v1)
        sems = (sem0, sem1)
        pltpu.sync_copy(idx_hbm.at[wid // (TOPK // BPW), pl.ds((wid % (TOPK // BPW)) * BPW, BPW)], idx_v)
        cur = pltpu.async_copy(table_hbm.at[idx_v.at[pl.ds(0, CH)]], bufs[0], sems[0])
        for c in range(NCH):
            nxt = None
            if c + 1 < NCH:
                nxt = pltpu.async_copy(table_hbm.at[idx_v.at[pl.ds((c + 1) * CH, CH)]],
                                       bufs[(c + 1) % 2], sems[(c + 1) % 2])
            cur.wait()
            pltpu.sync_copy(bufs[c % 2], out_hbm.at[pl.ds(base + c * CH, CH)])
            cur = nxt

    return gk(gidx, table)


def kernel(x, routing_token, num_tokens):
    b, n, d = x.shape
    x2 = x.reshape(NROWS, XD)
    nblk = NROWS // MV_BLK
    s3 = pl.pallas_call(
        _mv_body,
        grid=(nblk,),
        in_specs=[pl.BlockSpec((MV_BLK, XD), lambda i: (i, 0)),
                  pl.BlockSpec((1, XD), lambda i: (0, 0))],
        out_specs=pl.BlockSpec((1, 1, MV_BLK), lambda i: (i, 0, 0)),
        out_shape=jax.ShapeDtypeStruct((nblk, 1, MV_BLK), jnp.float32),
    )(x2, routing_token)
    s = s3.reshape(b, n)

    nt_arr = jnp.asarray(num_tokens, jnp.int32).reshape(1, 1)
    sel_i, sel_sc, gidx = _router_call(s, nt_arr)
    routed = _sc_gather(x2, gidx).reshape(b, TOPK, XD)
    return sel_i, sel_sc, routed
